# baseline (speedup 1.0000x reference)
import functools

import jax
import jax.numpy as jnp
from jax import lax
from jax.experimental import pallas as pl
from jax.experimental.pallas import tpu as pltpu
from jax.experimental.pallas import tpu_sc as plsc

_ENTITY = 100000
_REL = 1000
_EMB = 128
_IMG = 512
_B = 16384
_MULT = 20.0
_PSI = 1.0
_EPS = 1e-5

_NHALF = 2
_H = _B // _NHALF
_NW = 32
_ROWS_W = _H // _NW
_CH = 16
_NCH = _ROWS_W // _CH


def _sc_gather_body(s_hbm, o_hbm, r_hbm, e_hbm, r_t_hbm, rht_t_hbm, rtt_t_hbm,
                    img_hbm,
                    img_s_out, img_o_out, rht_out, rtt_out, base_out,
                    idx_s, idx_o, idx_r, base_buf,
                    bufs_a, bufs_b,
                    gs_a, gs_b, ws_a, ws_b):
    sid = lax.axis_index("s")
    wid = sid * 2 + lax.axis_index("c")
    tbase = wid * _ROWS_W

    pltpu.sync_copy(s_hbm.at[pl.ds(tbase, _ROWS_W)], idx_s)
    pltpu.sync_copy(o_hbm.at[pl.ds(tbase, _ROWS_W)], idx_o)
    pltpu.sync_copy(r_hbm.at[pl.ds(tbase, _ROWS_W)], idx_r)

    def fire_g(bufs, sem, c):
        off = c * _CH
        bs, bo, bes, beo, brr, brht, brtt = bufs
        isl = idx_s.at[pl.ds(off, _CH)]
        iol = idx_o.at[pl.ds(off, _CH)]
        irl = idx_r.at[pl.ds(off, _CH)]
        pltpu.async_copy(img_hbm.at[isl], bs, sem)
        pltpu.async_copy(img_hbm.at[iol], bo, sem)
        pltpu.async_copy(e_hbm.at[isl], bes, sem)
        pltpu.async_copy(e_hbm.at[iol], beo, sem)
        pltpu.async_copy(r_t_hbm.at[irl], brr, sem)
        pltpu.async_copy(rht_t_hbm.at[irl], brht, sem)
        pltpu.async_copy(rtt_t_hbm.at[irl], brtt, sem)

    def wait_g(bufs, sem):
        bs, bo, bes, beo, brr, brht, brtt = bufs
        dummy = pl.ds(0, _CH)
        pltpu.make_async_copy(img_hbm.at[dummy], bs, sem).wait()
        pltpu.make_async_copy(img_hbm.at[dummy], bo, sem).wait()
        pltpu.make_async_copy(e_hbm.at[dummy], bes, sem).wait()
        pltpu.make_async_copy(e_hbm.at[dummy], beo, sem).wait()
        pltpu.make_async_copy(r_t_hbm.at[dummy], brr, sem).wait()
        pltpu.make_async_copy(rht_t_hbm.at[dummy], brht, sem).wait()
        pltpu.make_async_copy(rtt_t_hbm.at[dummy], brtt, sem).wait()

    def fire_wb(bufs, sem, c):
        dst = tbase + c * _CH
        bs, bo, bes, beo, brr, brht, brtt = bufs
        pltpu.async_copy(bs, img_s_out.at[pl.ds(dst, _CH)], sem)
        pltpu.async_copy(bo, img_o_out.at[pl.ds(dst, _CH)], sem)
        pltpu.async_copy(brht, rht_out.at[pl.ds(dst, _CH)], sem)
        pltpu.async_copy(brtt, rtt_out.at[pl.ds(dst, _CH)], sem)

    def wait_wb(bufs, sem):
        bs, bo, bes, beo, brr, brht, brtt = bufs
        dummy = pl.ds(0, _CH)
        pltpu.make_async_copy(bs, img_s_out.at[dummy], sem).wait()
        pltpu.make_async_copy(bo, img_o_out.at[dummy], sem).wait()
        pltpu.make_async_copy(brht, rht_out.at[dummy], sem).wait()
        pltpu.make_async_copy(brtt, rtt_out.at[dummy], sem).wait()

    def compute_base(bufs, c):
        bs, bo, bes, beo, brr, brht, brtt = bufs

        def row_fn(j, carry):
            acc = jnp.zeros((16,), jnp.float32)
            for k in range(_EMB // 16):
                sl = pl.ds(k * 16, 16)
                acc = acc + bes[j, sl] * brr[j, sl] * beo[j, sl]
            row = c * _CH + j
            base_buf[row // 8, pl.ds((row % 8) * 16, 16)] = acc
            return carry

        lax.fori_loop(0, _CH, row_fn, 0)

    fire_g(bufs_a, gs_a, 0)

    def body(i, carry):
        c0 = 2 * i
        c1 = 2 * i + 1
        wait_g(bufs_a, gs_a)
        fire_wb(bufs_a, ws_a, c0)
        fire_g(bufs_b, gs_b, c1)
        compute_base(bufs_a, c0)
        wait_g(bufs_b, gs_b)
        fire_wb(bufs_b, ws_b, c1)
        wait_wb(bufs_a, ws_a)
        fire_g(bufs_a, gs_a, jnp.minimum(c0 + 2, _NCH - 1))
        compute_base(bufs_b, c1)
        wait_wb(bufs_b, ws_b)
        return carry

    lax.fori_loop(0, _NCH // 2, body, 0)
    wait_g(bufs_a, gs_a)
    pltpu.sync_copy(
        base_buf,
        base_out.at[pl.ds(pl.multiple_of(tbase // 8, 8), _ROWS_W // 8)])


def _buf_set():
    return (
        pltpu.VMEM((_CH, _IMG), jnp.float32),
        pltpu.VMEM((_CH, _IMG), jnp.float32),
        pltpu.VMEM((_CH, _EMB), jnp.float32),
        pltpu.VMEM((_CH, _EMB), jnp.float32),
        pltpu.VMEM((_CH, _EMB), jnp.float32),
        pltpu.VMEM((_CH, _EMB), jnp.float32),
        pltpu.VMEM((_CH, _EMB), jnp.float32),
    )


def _build_sc_gather():
  return functools.partial(
    pl.kernel,
    mesh=plsc.VectorSubcoreMesh(core_axis_name="c", subcore_axis_name="s"),
    out_type=(
        jax.ShapeDtypeStruct((_H, _IMG), jnp.float32),
        jax.ShapeDtypeStruct((_H, _IMG), jnp.float32),
        jax.ShapeDtypeStruct((_H, _EMB), jnp.float32),
        jax.ShapeDtypeStruct((_H, _EMB), jnp.float32),
        jax.ShapeDtypeStruct((_H // 8, _EMB), jnp.float32),
    ),
    scratch_types=[
        pltpu.VMEM((_ROWS_W,), jnp.int32),
        pltpu.VMEM((_ROWS_W,), jnp.int32),
        pltpu.VMEM((_ROWS_W,), jnp.int32),
        pltpu.VMEM((_ROWS_W // 8, _EMB), jnp.float32),
        _buf_set(),
        _buf_set(),
        pltpu.SemaphoreType.DMA,
        pltpu.SemaphoreType.DMA,
        pltpu.SemaphoreType.DMA,
        pltpu.SemaphoreType.DMA,
    ],
  )(_sc_gather_body)


_BLK = 1024


def _mm_body(img_s_ref, img_o_ref, wt_ref, b_ref, ps_ref, po_ref, st_ref,
             acc_ref):
    i = pl.program_id(0)
    wt = wt_ref[...].astype(jnp.bfloat16)
    ps = jnp.dot(img_s_ref[...].astype(jnp.bfloat16), wt,
                 preferred_element_type=jnp.float32) + b_ref[...]
    po = jnp.dot(img_o_ref[...].astype(jnp.bfloat16), wt,
                 preferred_element_type=jnp.float32) + b_ref[...]
    ps_ref[...] = ps
    po_ref[...] = po
    part = jnp.concatenate([
        jnp.sum(ps, axis=0, keepdims=True),
        jnp.sum(ps * ps, axis=0, keepdims=True),
        jnp.sum(po, axis=0, keepdims=True),
        jnp.sum(po * po, axis=0, keepdims=True),
    ], axis=0)

    @pl.when(i == 0)
    def _():
        acc_ref[...] = part

    @pl.when(i > 0)
    def _():
        acc_ref[...] += part

    @pl.when(i == pl.num_programs(0) - 1)
    def _():
        st_ref[...] = acc_ref[...]


def _tc_project(img_s, img_o, wt, b2):
    return pl.pallas_call(
        _mm_body,
        grid=(_H // _BLK,),
        in_specs=[
            pl.BlockSpec((_BLK, _IMG), lambda i: (i, 0)),
            pl.BlockSpec((_BLK, _IMG), lambda i: (i, 0)),
            pl.BlockSpec((_IMG, _EMB), lambda i: (0, 0)),
            pl.BlockSpec((1, _EMB), lambda i: (0, 0)),
        ],
        out_specs=[
            pl.BlockSpec((_BLK, _EMB), lambda i: (i, 0)),
            pl.BlockSpec((_BLK, _EMB), lambda i: (i, 0)),
            pl.BlockSpec((4, _EMB), lambda i: (0, 0)),
        ],
        out_shape=[
            jax.ShapeDtypeStruct((_H, _EMB), jnp.float32),
            jax.ShapeDtypeStruct((_H, _EMB), jnp.float32),
            jax.ShapeDtypeStruct((4, _EMB), jnp.float32),
        ],
        scratch_shapes=[pltpu.VMEM((4, _EMB), jnp.float32)],
    )(img_s, img_o, wt, b2)


def _final_body(ps_ref, po_ref, rht_ref, rtt_ref, base_ref, st0_ref, st1_ref,
                gamma_ref, beta_ref, out_ref):
    n = jnp.float32(_B)
    gamma = gamma_ref[...]
    beta = beta_ref[...]
    st = st0_ref[...] + st1_ref[...]

    mu_s = st[0:1, :] / n
    var_s = st[1:2, :] / n - mu_s * mu_s
    inv_s = lax.rsqrt(var_s + _EPS)
    mu_o = st[2:3, :] / n
    var_o = st[3:4, :] / n - mu_o * mu_o
    inv_o = lax.rsqrt(var_o + _EPS)

    s_img = gamma * (ps_ref[...] - mu_s) * inv_s + beta
    o_img = gamma * (po_ref[...] - mu_o) * inv_o + beta

    head = jax.nn.sigmoid(
        _PSI * jnp.sum(s_img * rht_ref[...], axis=-1, keepdims=True))
    tail = jax.nn.sigmoid(
        _PSI * jnp.sum(o_img * rtt_ref[...], axis=-1, keepdims=True))
    expand = (lax.broadcasted_iota(jnp.int32, (_BLK, _BLK // 8), 0) // 8
              == lax.broadcasted_iota(jnp.int32, (_BLK, _BLK // 8), 1)
              ).astype(jnp.float32)
    tmp = jnp.dot(expand, base_ref[...], preferred_element_type=jnp.float32)
    group = (lax.broadcasted_iota(jnp.int32, (_BLK, _EMB), 1) // 16
             == lax.broadcasted_iota(jnp.int32, (_BLK, _EMB), 0) % 8
             ).astype(jnp.float32)
    bsum = jnp.sum(tmp * group, axis=-1, keepdims=True)
    base = jax.nn.sigmoid(_PSI * bsum)
    out_ref[...] = _MULT * base * head * tail


def _tc_final(ps, po, rht, rtt, base2, st0, st1, gamma2, beta2):
    emb_spec = pl.BlockSpec((_BLK, _EMB), lambda i: (i, 0))
    return pl.pallas_call(
        _final_body,
        grid=(_H // _BLK,),
        in_specs=[
            emb_spec, emb_spec, emb_spec, emb_spec,
            pl.BlockSpec((_BLK // 8, _EMB), lambda i: (i, 0)),
            pl.BlockSpec((4, _EMB), lambda i: (0, 0)),
            pl.BlockSpec((4, _EMB), lambda i: (0, 0)),
            pl.BlockSpec((1, _EMB), lambda i: (0, 0)),
            pl.BlockSpec((1, _EMB), lambda i: (0, 0)),
        ],
        out_specs=pl.BlockSpec((_BLK, 1), lambda i: (i, 0)),
        out_shape=jax.ShapeDtypeStruct((_H, 1), jnp.float32),
    )(ps, po, rht, rtt, base2, st0, st1, gamma2, beta2)


def kernel(s, r, o, E, R, R_ht, R_tt, image_w, W, b, gamma, beta):
    s_flat = s.reshape(-1)
    r_flat = r.reshape(-1)
    o_flat = o.reshape(-1)
    wt = W.T
    b2 = b.reshape(1, _EMB)
    gamma2 = gamma.reshape(1, _EMB)
    beta2 = beta.reshape(1, _EMB)

    sc = _build_sc_gather()
    halves = []
    for h in range(_NHALF):
        sl = slice(h * _H, (h + 1) * _H)
        img_s, img_o, rht, rtt, basep = sc(
            s_flat[sl], o_flat[sl], r_flat[sl], E, R, R_ht, R_tt, image_w)
        ps, po, st = _tc_project(img_s, img_o, wt, b2)
        halves.append((ps, po, rht, rtt, basep, st))

    st0 = halves[0][5]
    st1 = (halves[1][5] if _NHALF > 1
           else jnp.zeros((4, _EMB), jnp.float32))
    outs = [
        _tc_final(ps, po, rht, rtt, basep, st0, st1, gamma2, beta2)
        for (ps, po, rht, rtt, basep, _) in halves
    ]
    return jnp.concatenate(outs, axis=0)


# speedup vs baseline: 1.1332x; 1.1332x over previous
import functools

import jax
import jax.numpy as jnp
from jax import lax
from jax.experimental import pallas as pl
from jax.experimental.pallas import tpu as pltpu
from jax.experimental.pallas import tpu_sc as plsc

_ENTITY = 100000
_REL = 1000
_EMB = 128
_IMG = 512
_B = 16384
_MULT = 20.0
_PSI = 1.0
_EPS = 1e-5

_NHALF = 1
_H = _B // _NHALF
_NW = 32
_ROWS_W = _H // _NW
_CH = 32
_NCH = _ROWS_W // _CH


def _sc_gather_body(s_hbm, o_hbm, r_hbm, e_hbm, r_t_hbm, rht_t_hbm, rtt_t_hbm,
                    img_hbm,
                    img_s_out, img_o_out, rht_out, rtt_out, base_out,
                    idx_s, idx_o, idx_r, base_buf,
                    bufs_a, bufs_b,
                    gs_a, gs_b, ws_a, ws_b):
    sid = lax.axis_index("s")
    wid = sid * 2 + lax.axis_index("c")
    tbase = wid * _ROWS_W

    pltpu.sync_copy(s_hbm.at[pl.ds(tbase, _ROWS_W)], idx_s)
    pltpu.sync_copy(o_hbm.at[pl.ds(tbase, _ROWS_W)], idx_o)
    pltpu.sync_copy(r_hbm.at[pl.ds(tbase, _ROWS_W)], idx_r)

    def fire_g(bufs, sem, c):
        off = c * _CH
        bs, bo, bes, beo, brr, brht, brtt = bufs
        isl = idx_s.at[pl.ds(off, _CH)]
        iol = idx_o.at[pl.ds(off, _CH)]
        irl = idx_r.at[pl.ds(off, _CH)]
        pltpu.async_copy(img_hbm.at[isl], bs, sem)
        pltpu.async_copy(img_hbm.at[iol], bo, sem)
        pltpu.async_copy(e_hbm.at[isl], bes, sem)
        pltpu.async_copy(e_hbm.at[iol], beo, sem)
        pltpu.async_copy(r_t_hbm.at[irl], brr, sem)
        pltpu.async_copy(rht_t_hbm.at[irl], brht, sem)
        pltpu.async_copy(rtt_t_hbm.at[irl], brtt, sem)

    def wait_g(bufs, sem):
        bs, bo, bes, beo, brr, brht, brtt = bufs
        dummy = pl.ds(0, _CH)
        pltpu.make_async_copy(img_hbm.at[dummy], bs, sem).wait()
        pltpu.make_async_copy(img_hbm.at[dummy], bo, sem).wait()
        pltpu.make_async_copy(e_hbm.at[dummy], bes, sem).wait()
        pltpu.make_async_copy(e_hbm.at[dummy], beo, sem).wait()
        pltpu.make_async_copy(r_t_hbm.at[dummy], brr, sem).wait()
        pltpu.make_async_copy(rht_t_hbm.at[dummy], brht, sem).wait()
        pltpu.make_async_copy(rtt_t_hbm.at[dummy], brtt, sem).wait()

    def fire_wb(bufs, sem, c):
        dst = tbase + c * _CH
        bs, bo, bes, beo, brr, brht, brtt = bufs
        pltpu.async_copy(bs, img_s_out.at[pl.ds(dst, _CH)], sem)
        pltpu.async_copy(bo, img_o_out.at[pl.ds(dst, _CH)], sem)
        pltpu.async_copy(brht, rht_out.at[pl.ds(dst, _CH)], sem)
        pltpu.async_copy(brtt, rtt_out.at[pl.ds(dst, _CH)], sem)

    def wait_wb(bufs, sem):
        bs, bo, bes, beo, brr, brht, brtt = bufs
        dummy = pl.ds(0, _CH)
        pltpu.make_async_copy(bs, img_s_out.at[dummy], sem).wait()
        pltpu.make_async_copy(bo, img_o_out.at[dummy], sem).wait()
        pltpu.make_async_copy(brht, rht_out.at[dummy], sem).wait()
        pltpu.make_async_copy(brtt, rtt_out.at[dummy], sem).wait()

    def compute_base(bufs, c):
        bs, bo, bes, beo, brr, brht, brtt = bufs

        def row_fn(j, carry):
            acc = jnp.zeros((16,), jnp.float32)
            for k in range(_EMB // 16):
                sl = pl.ds(k * 16, 16)
                acc = acc + bes[j, sl] * brr[j, sl] * beo[j, sl]
            row = c * _CH + j
            base_buf[row // 8, pl.ds((row % 8) * 16, 16)] = acc
            return carry

        lax.fori_loop(0, _CH, row_fn, 0)

    fire_g(bufs_a, gs_a, 0)

    def body(i, carry):
        c0 = 2 * i
        c1 = 2 * i + 1
        wait_g(bufs_a, gs_a)
        fire_wb(bufs_a, ws_a, c0)
        fire_g(bufs_b, gs_b, c1)
        compute_base(bufs_a, c0)
        wait_g(bufs_b, gs_b)
        fire_wb(bufs_b, ws_b, c1)
        wait_wb(bufs_a, ws_a)
        fire_g(bufs_a, gs_a, jnp.minimum(c0 + 2, _NCH - 1))
        compute_base(bufs_b, c1)
        wait_wb(bufs_b, ws_b)
        return carry

    lax.fori_loop(0, _NCH // 2, body, 0)
    wait_g(bufs_a, gs_a)
    pltpu.sync_copy(
        base_buf,
        base_out.at[pl.ds(pl.multiple_of(tbase // 8, 8), _ROWS_W // 8)])


def _buf_set():
    return (
        pltpu.VMEM((_CH, _IMG), jnp.float32),
        pltpu.VMEM((_CH, _IMG), jnp.float32),
        pltpu.VMEM((_CH, _EMB), jnp.float32),
        pltpu.VMEM((_CH, _EMB), jnp.float32),
        pltpu.VMEM((_CH, _EMB), jnp.float32),
        pltpu.VMEM((_CH, _EMB), jnp.float32),
        pltpu.VMEM((_CH, _EMB), jnp.float32),
    )


def _build_sc_gather():
  return functools.partial(
    pl.kernel,
    mesh=plsc.VectorSubcoreMesh(core_axis_name="c", subcore_axis_name="s"),
    out_type=(
        jax.ShapeDtypeStruct((_H, _IMG), jnp.float32),
        jax.ShapeDtypeStruct((_H, _IMG), jnp.float32),
        jax.ShapeDtypeStruct((_H, _EMB), jnp.float32),
        jax.ShapeDtypeStruct((_H, _EMB), jnp.float32),
        jax.ShapeDtypeStruct((_H // 8, _EMB), jnp.float32),
    ),
    scratch_types=[
        pltpu.VMEM((_ROWS_W,), jnp.int32),
        pltpu.VMEM((_ROWS_W,), jnp.int32),
        pltpu.VMEM((_ROWS_W,), jnp.int32),
        pltpu.VMEM((_ROWS_W // 8, _EMB), jnp.float32),
        _buf_set(),
        _buf_set(),
        pltpu.SemaphoreType.DMA,
        pltpu.SemaphoreType.DMA,
        pltpu.SemaphoreType.DMA,
        pltpu.SemaphoreType.DMA,
    ],
  )(_sc_gather_body)


_BLK = 2048


def _mm_body(img_s_ref, img_o_ref, wt_ref, b_ref, ps_ref, po_ref, st_ref,
             acc_ref):
    i = pl.program_id(0)
    wt = wt_ref[...].astype(jnp.bfloat16)
    ps = jnp.dot(img_s_ref[...].astype(jnp.bfloat16), wt,
                 preferred_element_type=jnp.float32) + b_ref[...]
    po = jnp.dot(img_o_ref[...].astype(jnp.bfloat16), wt,
                 preferred_element_type=jnp.float32) + b_ref[...]
    ps_ref[...] = ps
    po_ref[...] = po
    part = jnp.concatenate([
        jnp.sum(ps, axis=0, keepdims=True),
        jnp.sum(ps * ps, axis=0, keepdims=True),
        jnp.sum(po, axis=0, keepdims=True),
        jnp.sum(po * po, axis=0, keepdims=True),
    ], axis=0)

    @pl.when(i == 0)
    def _():
        acc_ref[...] = part

    @pl.when(i > 0)
    def _():
        acc_ref[...] += part

    @pl.when(i == pl.num_programs(0) - 1)
    def _():
        st_ref[...] = acc_ref[...]


def _tc_project(img_s, img_o, wt, b2):
    return pl.pallas_call(
        _mm_body,
        grid=(_H // _BLK,),
        in_specs=[
            pl.BlockSpec((_BLK, _IMG), lambda i: (i, 0)),
            pl.BlockSpec((_BLK, _IMG), lambda i: (i, 0)),
            pl.BlockSpec((_IMG, _EMB), lambda i: (0, 0)),
            pl.BlockSpec((1, _EMB), lambda i: (0, 0)),
        ],
        out_specs=[
            pl.BlockSpec((_BLK, _EMB), lambda i: (i, 0)),
            pl.BlockSpec((_BLK, _EMB), lambda i: (i, 0)),
            pl.BlockSpec((4, _EMB), lambda i: (0, 0)),
        ],
        out_shape=[
            jax.ShapeDtypeStruct((_H, _EMB), jnp.float32),
            jax.ShapeDtypeStruct((_H, _EMB), jnp.float32),
            jax.ShapeDtypeStruct((4, _EMB), jnp.float32),
        ],
        scratch_shapes=[pltpu.VMEM((4, _EMB), jnp.float32)],
    )(img_s, img_o, wt, b2)


def _final_body(ps_ref, po_ref, rht_ref, rtt_ref, base_ref, st0_ref, st1_ref,
                gamma_ref, beta_ref, out_ref):
    n = jnp.float32(_B)
    gamma = gamma_ref[...]
    beta = beta_ref[...]
    st = st0_ref[...] + st1_ref[...]

    mu_s = st[0:1, :] / n
    var_s = st[1:2, :] / n - mu_s * mu_s
    inv_s = lax.rsqrt(var_s + _EPS)
    mu_o = st[2:3, :] / n
    var_o = st[3:4, :] / n - mu_o * mu_o
    inv_o = lax.rsqrt(var_o + _EPS)

    s_img = gamma * (ps_ref[...] - mu_s) * inv_s + beta
    o_img = gamma * (po_ref[...] - mu_o) * inv_o + beta

    head = jax.nn.sigmoid(
        _PSI * jnp.sum(s_img * rht_ref[...], axis=-1, keepdims=True))
    tail = jax.nn.sigmoid(
        _PSI * jnp.sum(o_img * rtt_ref[...], axis=-1, keepdims=True))
    expand = (lax.broadcasted_iota(jnp.int32, (_BLK, _BLK // 8), 0) // 8
              == lax.broadcasted_iota(jnp.int32, (_BLK, _BLK // 8), 1)
              ).astype(jnp.float32)
    tmp = jnp.dot(expand, base_ref[...], preferred_element_type=jnp.float32)
    group = (lax.broadcasted_iota(jnp.int32, (_BLK, _EMB), 1) // 16
             == lax.broadcasted_iota(jnp.int32, (_BLK, _EMB), 0) % 8
             ).astype(jnp.float32)
    bsum = jnp.sum(tmp * group, axis=-1, keepdims=True)
    base = jax.nn.sigmoid(_PSI * bsum)
    out_ref[...] = _MULT * base * head * tail


def _tc_final(ps, po, rht, rtt, base2, st0, st1, gamma2, beta2):
    emb_spec = pl.BlockSpec((_BLK, _EMB), lambda i: (i, 0))
    return pl.pallas_call(
        _final_body,
        grid=(_H // _BLK,),
        in_specs=[
            emb_spec, emb_spec, emb_spec, emb_spec,
            pl.BlockSpec((_BLK // 8, _EMB), lambda i: (i, 0)),
            pl.BlockSpec((4, _EMB), lambda i: (0, 0)),
            pl.BlockSpec((4, _EMB), lambda i: (0, 0)),
            pl.BlockSpec((1, _EMB), lambda i: (0, 0)),
            pl.BlockSpec((1, _EMB), lambda i: (0, 0)),
        ],
        out_specs=pl.BlockSpec((_BLK, 1), lambda i: (i, 0)),
        out_shape=jax.ShapeDtypeStruct((_H, 1), jnp.float32),
    )(ps, po, rht, rtt, base2, st0, st1, gamma2, beta2)


def kernel(s, r, o, E, R, R_ht, R_tt, image_w, W, b, gamma, beta):
    s_flat = s.reshape(-1)
    r_flat = r.reshape(-1)
    o_flat = o.reshape(-1)
    wt = W.T
    b2 = b.reshape(1, _EMB)
    gamma2 = gamma.reshape(1, _EMB)
    beta2 = beta.reshape(1, _EMB)

    sc = _build_sc_gather()
    halves = []
    for h in range(_NHALF):
        sl = slice(h * _H, (h + 1) * _H)
        img_s, img_o, rht, rtt, basep = sc(
            s_flat[sl], o_flat[sl], r_flat[sl], E, R, R_ht, R_tt, image_w)
        ps, po, st = _tc_project(img_s, img_o, wt, b2)
        halves.append((ps, po, rht, rtt, basep, st))

    st0 = halves[0][5]
    st1 = (halves[1][5] if _NHALF > 1
           else jnp.zeros((4, _EMB), jnp.float32))
    outs = [
        _tc_final(ps, po, rht, rtt, basep, st0, st1, gamma2, beta2)
        for (ps, po, rht, rtt, basep, _) in halves
    ]
    return jnp.concatenate(outs, axis=0)
